# baseline (speedup 1.0000x reference)
import functools

import jax
import jax.numpy as jnp
from jax.experimental import pallas as pl

_CH = 8


def _body(inp_ref, dir_ref, prob_ref, out_ref):
    G, H, W = out_ref.shape
    n = H // _CH

    jj = jax.lax.broadcasted_iota(jnp.int32, (_CH, W), 1)
    col_l, col_r = jj >= 1, jj < W - 1
    fill_col = jnp.full((_CH, 1), -1, jnp.int32)
    fill_row = jnp.full((1, W), -1, jnp.int32)
    true2 = jnp.full((_CH, W), True)

    def colL(x):
        return jnp.concatenate([x[:, 1:2], x[:, :-1]], axis=1)

    def colR(x):
        return jnp.concatenate([x[:, 1:], x[:, -2:-1]], axis=1)

    def colLm(x):
        return jnp.concatenate([fill_col, x[:, :-1]], axis=1)

    def colRm(x):
        return jnp.concatenate([x[:, 1:], fill_col], axis=1)

    for g in range(G):
        _img(inp_ref, dir_ref, prob_ref, out_ref, g, n,
             col_l, col_r, fill_col, fill_row, true2, colL, colR, colLm, colRm)


def _img(inp_ref, dir_ref, prob_ref, out_ref, g, n,
         col_l, col_r, fill_col, fill_row, true2, colL, colR, colLm, colRm):
    _CH_ = _CH
    H = out_ref.shape[1]
    W = out_ref.shape[2]

    def ld(k):
        s = slice(k * _CH, (k + 1) * _CH)
        a = inp_ref[g, s, :]
        e = jnp.where(prob_ref[g, s, :] <= 20, dir_ref[g, s, :], -1)
        return a, e

    a_p = e_p = a_n = e_n = None
    a_c, e_c = ld(0)
    for k in range(n):
        if k + 1 < n:
            a_n, e_n = ld(k + 1)
        if k == 0:
            up = jnp.concatenate([a_c[1:2], a_c[:_CH - 1]], axis=0)
            eu = jnp.concatenate([fill_row, e_c[:_CH - 1]], axis=0)
        else:
            up = jnp.concatenate([a_p[_CH - 1:], a_c[:_CH - 1]], axis=0)
            eu = jnp.concatenate([e_p[_CH - 1:], e_c[:_CH - 1]], axis=0)
        if k == n - 1:
            dn = jnp.concatenate([a_c[1:], a_c[_CH - 2:_CH - 1]], axis=0)
            ed = jnp.concatenate([e_c[1:], fill_row], axis=0)
        else:
            dn = jnp.concatenate([a_c[1:], a_n[:1]], axis=0)
            ed = jnp.concatenate([e_c[1:], e_n[:1]], axis=0)

        rs = up + a_c + dn
        avg = (colL(rs) + rs + colR(rs)) * (1.0 / 9.0)

        si = {0: colR(dn), 1: dn, 2: colL(dn), 3: colR(a_c),
              5: colL(a_c), 6: colR(up), 7: up}
        se = {0: colRm(ed), 1: ed, 2: colLm(ed), 3: colRm(e_c),
              5: colLm(e_c), 6: colRm(eu), 7: eu}

        if k == 0:
            ii = jax.lax.broadcasted_iota(jnp.int32, (_CH, W), 0)
            row_up = ii >= 1
        else:
            row_up = true2
        if k == n - 1:
            ii = jax.lax.broadcasted_iota(jnp.int32, (_CH, W), 0)
            row_dn = ii < _CH - 1
        else:
            row_dn = true2
        inb = {0: row_up & col_l, 1: row_up, 2: row_up & col_r, 3: col_l,
               5: col_r, 6: row_dn & col_l, 7: row_dn}

        rules = []
        for d in range(8):
            if d == 4:
                rules.append((e_c == 4, avg))
                continue
            rules.append((se[d] == d, si[d]))
            rules.append(((e_c == d) & inb[d], avg))

        def fold(seg):
            y = a_c
            any_c = None
            for c, v in seg:
                y = jnp.where(c, v, y)
                any_c = c if any_c is None else (any_c | c)
            return y, any_c

        x, _ = fold(rules[0:5])
        y2, any2 = fold(rules[5:10])
        y3, any3 = fold(rules[10:15])
        x = jnp.where(any2, y2, x)
        x = jnp.where(any3, y3, x)
        out_ref[g, k * _CH:(k + 1) * _CH, :] = x

        a_p, e_p = a_c, e_c
        a_c, e_c = a_n, e_n


@functools.partial(jax.jit, static_argnames=("interpret",))
def kernel(inp, direction, prob, interpret=False):
    B, C, H, W = inp.shape
    N = B * C
    a3 = inp.reshape(N, H, W)
    d3 = direction.reshape(N, H, W)
    p3 = prob.reshape(N, H, W)
    G = 8
    spec = pl.BlockSpec((G, H, W), lambda i: (i, 0, 0))
    out = pl.pallas_call(
        _body,
        grid=(N // G,),
        in_specs=[spec, spec, spec],
        out_specs=spec,
        out_shape=jax.ShapeDtypeStruct((N, H, W), inp.dtype),
        interpret=interpret,
    )(a3, d3, p3)
    return out.reshape(B, C, H, W)
